# baseline (speedup 1.0000x reference)
import functools
import math

import jax
import jax.numpy as jnp
import numpy as np
from jax import lax
from jax.experimental import pallas as pl
from jax.experimental.pallas import tpu as pltpu
from jax.experimental.pallas import tpu_sc as plsc

B, T, C_IN, D_MODEL = 16, 2048, 16, 128
HALF = T // 2
MBINS = HALF // 2
CHUNK = 128
NCHUNK = MBINS // CHUNK
HI = jax.lax.Precision.HIGHEST
MED = jax.lax.Precision.DEFAULT


def _sinusoid_table(rows, d_model):
    pos = np.arange(rows, dtype=np.float32)[:, None]
    div = np.exp(np.arange(0, d_model, 2, dtype=np.float32)
                 * -(math.log(10000.0) / d_model))
    w = np.zeros((rows, d_model), dtype=np.float32)
    w[:, 0::2] = np.sin(pos * div)
    w[:, 1::2] = np.cos(pos * div)
    return w


_PE = _sinusoid_table(T, D_MODEL)
_TAB8 = _sinusoid_table(8, D_MODEL)
_tt = np.arange(HALF, dtype=np.float64)[:, None]
_mm = np.arange(MBINS, dtype=np.float64)[None, :]
_CE = np.cos(2.0 * np.pi * _tt * _mm / HALF).astype(np.float32)
_SE = np.sin(2.0 * np.pi * _tt * _mm / HALF).astype(np.float32)
_CO = np.cos(2.0 * np.pi * _tt * (2.0 * _mm + 1.0) / T).astype(np.float32)
_SO = np.sin(2.0 * np.pi * _tt * (2.0 * _mm + 1.0) / T).astype(np.float32)
_TABS = np.stack([
    np.concatenate([t[:, c * CHUNK:(c + 1) * CHUNK]
                    for t in (_CE, _SE, _CO, _SO)], axis=1)
    for c in range(NCHUNK)])
_SEL = (np.arange(B)[:, None] ==
        (np.arange(B * C_IN)[None, :] // C_IN)).astype(np.float32)


_NWORKERS = 32
_TOK_PER_W = B * T // _NWORKERS
_ITERS = _TOK_PER_W // 16


def _counts_sc_body(xm_hbm, out_hbm, xm_v, cnt_v):
    wid = lax.axis_index("s") * 2 + lax.axis_index("c")
    base = wid * _TOK_PER_W
    pltpu.sync_copy(xm_hbm.at[:, pl.ds(base, _TOK_PER_W)], xm_v)
    iota = lax.broadcasted_iota(jnp.int32, (16,), 0)
    ones = jnp.full((16,), 1.0, dtype=jnp.float32)
    zeros = jnp.zeros((16,), dtype=jnp.float32)

    def body(it, _):
        tok = iota + it * 16
        for j in range(8):
            plsc.store_scatter(cnt_v, [jnp.full((16,), j, jnp.int32), tok],
                               zeros)
        for i in range(4):
            vi = plsc.load_gather(xm_v, [jnp.full((16,), i, jnp.int32), tok])
            plsc.addupdate_scatter(cnt_v, [vi, tok], ones)
        return _

    lax.fori_loop(0, _ITERS, body, 0)
    pltpu.sync_copy(cnt_v, out_hbm.at[:, pl.ds(base, _TOK_PER_W)])


@functools.cache
def _counts_sc():
    return pl.kernel(
        _counts_sc_body,
        mesh=plsc.VectorSubcoreMesh(core_axis_name="c", subcore_axis_name="s"),
        out_type=jax.ShapeDtypeStruct((8, B * T), jnp.float32),
        scratch_types=[
            pltpu.VMEM((4, _TOK_PER_W), jnp.int32),
            pltpu.VMEM((8, _TOK_PER_W), jnp.float32),
        ],
        compiler_params=pltpu.CompilerParams(needs_layout_passes=False),
    )


def _fused_body(xt_ref, tabs_ref, sel_ref,
                x_ref, cnt_ref, w_ref, tab8_ref, pe_ref,
                out_ref, alpha_ref, beta_ref, runmax_ref):
    c = pl.program_id(0)

    @pl.when(c < NCHUNK)
    def _():
        xt = xt_ref[...]
        xe = xt[:, :HALF] + xt[:, HALF:]
        xo = xt[:, :HALF] - xt[:, HALF:]
        tabs = tabs_ref[0]
        ree = jax.lax.dot(xe, tabs[:, :CHUNK], precision=HI)
        ime = jax.lax.dot(xe, tabs[:, CHUNK:2 * CHUNK], precision=HI)
        reo = jax.lax.dot(xo, tabs[:, 2 * CHUNK:3 * CHUNK], precision=HI)
        imo = jax.lax.dot(xo, tabs[:, 3 * CHUNK:], precision=HI)
        mag2 = jnp.maximum(ree * ree + ime * ime, reo * reo + imo * imo)
        chmax = jnp.max(mag2, axis=1, keepdims=True)
        chmax = jax.lax.broadcast_in_dim(chmax, (B * C_IN, D_MODEL), (0, 1))

        @pl.when(c == 0)
        def _():
            runmax_ref[...] = chmax

        @pl.when(c != 0)
        def _():
            runmax_ref[...] = jnp.maximum(runmax_ref[...], chmax)

        @pl.when(c == NCHUNK - 1)
        def _():
            ti = jax.lax.broadcasted_iota(jnp.int32, (1, HALF), 1)
            alt = (1 - 2 * (ti % 2)).astype(jnp.float32)
            nyqre = jnp.sum(xe * alt, axis=1, keepdims=True)
            nyq2 = nyqre * nyqre
            is_nyq = (nyq2 > runmax_ref[:, 0:1]).astype(jnp.float32)
            count = jax.lax.dot(sel_ref[...], is_nyq, precision=HI)
            beta = count * (1.0 / C_IN)
            beta_ref[...] = jax.lax.broadcast_in_dim(beta, (B, D_MODEL), (0, 1))
            alpha_ref[...] = 1.0 - beta_ref[...]

    @pl.when(c >= NCHUNK)
    def _():
        b = c - NCHUNK
        xb = x_ref[0]
        xprev = jnp.concatenate([xb[-1:], xb[:-1]], axis=0)
        xnext = jnp.concatenate([xb[1:], xb[:1]], axis=0)
        x3 = jnp.concatenate([xprev, xb, xnext], axis=1)
        val = jax.lax.dot(x3, w_ref[...], precision=MED)

        cnt = cnt_ref[...]
        temporal = jax.lax.dot_general(
            cnt, tab8_ref[...], (((0,), (0,)), ((), ())),
            precision=MED)

        a = alpha_ref[pl.ds(b, 1), :]
        b0 = beta_ref[pl.ds(b, 1), :] * pe_ref[0:1, :]
        out_ref[0] = val + temporal + a * pe_ref[...] + b0


def kernel(x, x_mark, W_conv):
    xt = jnp.transpose(x, (0, 2, 1)).reshape(B * C_IN, T)
    wt = jnp.transpose(W_conv, (2, 1, 0)).reshape(3 * C_IN, D_MODEL)

    xmT = jnp.transpose(x_mark.reshape(B * T, 4))
    counts = _counts_sc()(xmT)

    out = pl.pallas_call(
        _fused_body,
        grid=(NCHUNK + B,),
        in_specs=[
            pl.BlockSpec((B * C_IN, T), lambda c: (0, 0)),
            pl.BlockSpec((1, HALF, 4 * CHUNK),
                         lambda c: (jnp.minimum(c, NCHUNK - 1), 0, 0)),
            pl.BlockSpec((B, B * C_IN), lambda c: (0, 0)),
            pl.BlockSpec((1, T, C_IN),
                         lambda c: (jnp.maximum(c - NCHUNK, 0), 0, 0)),
            pl.BlockSpec((8, T), lambda c: (0, jnp.maximum(c - NCHUNK, 0))),
            pl.BlockSpec((3 * C_IN, D_MODEL), lambda c: (0, 0)),
            pl.BlockSpec((8, D_MODEL), lambda c: (0, 0)),
            pl.BlockSpec((T, D_MODEL), lambda c: (0, 0)),
        ],
        out_specs=pl.BlockSpec((1, T, D_MODEL),
                               lambda c: (jnp.maximum(c - NCHUNK, 0), 0, 0)),
        out_shape=jax.ShapeDtypeStruct((B, T, D_MODEL), jnp.float32),
        scratch_shapes=[
            pltpu.VMEM((B, D_MODEL), jnp.float32),
            pltpu.VMEM((B, D_MODEL), jnp.float32),
            pltpu.VMEM((B * C_IN, D_MODEL), jnp.float32),
        ],
    )(xt, jnp.asarray(_TABS), jnp.asarray(_SEL),
      x, counts, wt, jnp.asarray(_TAB8), jnp.asarray(_PE))
    return out


# speedup vs baseline: 1.0876x; 1.0876x over previous
import functools
import math

import jax
import jax.numpy as jnp
import numpy as np
from jax import lax
from jax.experimental import pallas as pl
from jax.experimental.pallas import tpu as pltpu
from jax.experimental.pallas import tpu_sc as plsc

B, T, C_IN, D_MODEL = 16, 2048, 16, 128
HALF = T // 2
MBINS = HALF // 2
CHUNK = 128
NCHUNK = MBINS // CHUNK
HI = jax.lax.Precision.HIGHEST
MED = jax.lax.Precision.DEFAULT


def _sinusoid_table(rows, d_model):
    pos = np.arange(rows, dtype=np.float32)[:, None]
    div = np.exp(np.arange(0, d_model, 2, dtype=np.float32)
                 * -(math.log(10000.0) / d_model))
    w = np.zeros((rows, d_model), dtype=np.float32)
    w[:, 0::2] = np.sin(pos * div)
    w[:, 1::2] = np.cos(pos * div)
    return w


_PE = _sinusoid_table(T, D_MODEL)
_TAB8 = _sinusoid_table(8, D_MODEL)
_tt = np.arange(HALF, dtype=np.float64)[:, None]
_mm = np.arange(MBINS, dtype=np.float64)[None, :]
_CO = np.cos(2.0 * np.pi * _tt * (2.0 * _mm + 1.0) / T).astype(np.float32)
_SO = np.sin(2.0 * np.pi * _tt * (2.0 * _mm + 1.0) / T).astype(np.float32)
_t2 = np.arange(HALF // 2, dtype=np.float64)[:, None]
_kk = np.arange(MBINS // 2, dtype=np.float64)[None, :]
_CEE = np.cos(2.0 * np.pi * _t2 * _kk / (HALF // 2)).astype(np.float32)
_SEE = np.sin(2.0 * np.pi * _t2 * _kk / (HALF // 2)).astype(np.float32)
_CEO = np.cos(2.0 * np.pi * _t2 * (2.0 * _kk + 1.0) / HALF).astype(np.float32)
_SEO = np.sin(2.0 * np.pi * _t2 * (2.0 * _kk + 1.0) / HALF).astype(np.float32)
_TABS = np.stack([
    np.concatenate([t[:, c * CHUNK:(c + 1) * CHUNK] for t in (_CO, _SO)],
                   axis=1)
    for c in range(NCHUNK)])
_EPACK = np.concatenate([_CEE, _SEE, _CEO, _SEO], axis=1)
_SEL = (np.arange(B)[:, None] ==
        (np.arange(B * C_IN)[None, :] // C_IN)).astype(np.float32)


_NWORKERS = 32
_TOK_PER_W = B * T // _NWORKERS
_ITERS = _TOK_PER_W // 16


def _counts_sc_body(xm_hbm, out_hbm, xm_v, cnt_v):
    wid = lax.axis_index("s") * 2 + lax.axis_index("c")
    base = wid * _TOK_PER_W
    pltpu.sync_copy(xm_hbm.at[:, pl.ds(base, _TOK_PER_W)], xm_v)
    iota = lax.broadcasted_iota(jnp.int32, (16,), 0)
    ones = jnp.full((16,), 1.0, dtype=jnp.float32)
    zeros = jnp.zeros((16,), dtype=jnp.float32)

    def body(it, _):
        tok = iota + it * 16
        for j in range(8):
            plsc.store_scatter(cnt_v, [jnp.full((16,), j, jnp.int32), tok],
                               zeros)
        for i in range(4):
            vi = plsc.load_gather(xm_v, [jnp.full((16,), i, jnp.int32), tok])
            plsc.addupdate_scatter(cnt_v, [vi, tok], ones)
        return _

    lax.fori_loop(0, _ITERS, body, 0)
    pltpu.sync_copy(cnt_v, out_hbm.at[:, pl.ds(base, _TOK_PER_W)])


@functools.cache
def _counts_sc():
    return pl.kernel(
        _counts_sc_body,
        mesh=plsc.VectorSubcoreMesh(core_axis_name="c", subcore_axis_name="s"),
        out_type=jax.ShapeDtypeStruct((8, B * T), jnp.float32),
        scratch_types=[
            pltpu.VMEM((4, _TOK_PER_W), jnp.int32),
            pltpu.VMEM((8, _TOK_PER_W), jnp.float32),
        ],
        compiler_params=pltpu.CompilerParams(needs_layout_passes=False),
    )


def _fused_body(xt_ref, tabs_ref, ep_ref, sel_ref,
                x_ref, cnt_ref, w_ref, tab8_ref, pe_ref,
                out_ref, alpha_ref, beta_ref, runmax_ref):
    c = pl.program_id(0)
    Q = HALF // 2

    @pl.when(c < NCHUNK)
    def _():
        xt = xt_ref[...]
        xe = xt[:, :HALF] + xt[:, HALF:]
        xo = xt[:, :HALF] - xt[:, HALF:]
        tabs = tabs_ref[0]
        reo = jax.lax.dot(xo, tabs[:, :CHUNK], precision=HI)
        imo = jax.lax.dot(xo, tabs[:, CHUNK:], precision=HI)
        mag2 = reo * reo + imo * imo
        chmax = jnp.max(mag2, axis=1, keepdims=True)
        chmax = jax.lax.broadcast_in_dim(chmax, (B * C_IN, D_MODEL), (0, 1))

        @pl.when(c == 0)
        def _():
            K2 = MBINS // 2
            xee = xe[:, :Q] + xe[:, Q:]
            xeo = xe[:, :Q] - xe[:, Q:]
            ep = ep_ref[...]
            ree = jax.lax.dot(xee, ep[:, :K2], precision=HI)
            ime = jax.lax.dot(xee, ep[:, K2:2 * K2], precision=HI)
            re2 = jax.lax.dot(xeo, ep[:, 2 * K2:3 * K2], precision=HI)
            im2 = jax.lax.dot(xeo, ep[:, 3 * K2:], precision=HI)
            emag = jnp.maximum(ree * ree + ime * ime, re2 * re2 + im2 * im2)
            echmax = jnp.max(emag, axis=1, keepdims=True)
            echmax = jax.lax.broadcast_in_dim(
                echmax, (B * C_IN, D_MODEL), (0, 1))
            runmax_ref[...] = jnp.maximum(chmax, echmax)

        @pl.when(c != 0)
        def _():
            runmax_ref[...] = jnp.maximum(runmax_ref[...], chmax)

        @pl.when(c == NCHUNK - 1)
        def _():
            ti = jax.lax.broadcasted_iota(jnp.int32, (1, HALF), 1)
            alt = (1 - 2 * (ti % 2)).astype(jnp.float32)
            nyqre = jnp.sum(xe * alt, axis=1, keepdims=True)
            nyq2 = nyqre * nyqre
            is_nyq = (nyq2 > runmax_ref[:, 0:1]).astype(jnp.float32)
            count = jax.lax.dot(sel_ref[...], is_nyq, precision=HI)
            beta = count * (1.0 / C_IN)
            beta_ref[...] = jax.lax.broadcast_in_dim(beta, (B, D_MODEL), (0, 1))
            alpha_ref[...] = 1.0 - beta_ref[...]

    @pl.when(c >= NCHUNK)
    def _():
        b = c - NCHUNK
        xb = x_ref[0]
        xprev = jnp.concatenate([xb[-1:], xb[:-1]], axis=0)
        xnext = jnp.concatenate([xb[1:], xb[:1]], axis=0)
        x3 = jnp.concatenate([xprev, xb, xnext], axis=1)
        val = jax.lax.dot(x3, w_ref[...], precision=MED)

        cnt = cnt_ref[...]
        temporal = jax.lax.dot_general(
            cnt, tab8_ref[...], (((0,), (0,)), ((), ())),
            precision=MED)

        a = alpha_ref[pl.ds(b, 1), :]
        b0 = beta_ref[pl.ds(b, 1), :] * pe_ref[0:1, :]
        out_ref[0] = val + temporal + a * pe_ref[...] + b0


def kernel(x, x_mark, W_conv):
    xt = jnp.transpose(x, (0, 2, 1)).reshape(B * C_IN, T)
    wt = jnp.transpose(W_conv, (2, 1, 0)).reshape(3 * C_IN, D_MODEL)

    xmT = jnp.transpose(x_mark.reshape(B * T, 4))
    counts = _counts_sc()(xmT)

    out = pl.pallas_call(
        _fused_body,
        grid=(NCHUNK + B,),
        in_specs=[
            pl.BlockSpec((B * C_IN, T), lambda c: (0, 0)),
            pl.BlockSpec((1, HALF, 2 * CHUNK),
                         lambda c: (jnp.minimum(c, NCHUNK - 1), 0, 0)),
            pl.BlockSpec((HALF // 2, 2 * MBINS), lambda c: (0, 0)),
            pl.BlockSpec((B, B * C_IN), lambda c: (0, 0)),
            pl.BlockSpec((1, T, C_IN),
                         lambda c: (jnp.maximum(c - NCHUNK, 0), 0, 0)),
            pl.BlockSpec((8, T), lambda c: (0, jnp.maximum(c - NCHUNK, 0))),
            pl.BlockSpec((3 * C_IN, D_MODEL), lambda c: (0, 0)),
            pl.BlockSpec((8, D_MODEL), lambda c: (0, 0)),
            pl.BlockSpec((T, D_MODEL), lambda c: (0, 0)),
        ],
        out_specs=pl.BlockSpec((1, T, D_MODEL),
                               lambda c: (jnp.maximum(c - NCHUNK, 0), 0, 0)),
        out_shape=jax.ShapeDtypeStruct((B, T, D_MODEL), jnp.float32),
        scratch_shapes=[
            pltpu.VMEM((B, D_MODEL), jnp.float32),
            pltpu.VMEM((B, D_MODEL), jnp.float32),
            pltpu.VMEM((B * C_IN, D_MODEL), jnp.float32),
        ],
    )(xt, jnp.asarray(_TABS), jnp.asarray(_EPACK), jnp.asarray(_SEL),
      x, counts, wt, jnp.asarray(_TAB8), jnp.asarray(_PE))
    return out


# speedup vs baseline: 1.2003x; 1.1037x over previous
import functools
import math

import jax
import jax.numpy as jnp
import numpy as np
from jax import lax
from jax.experimental import pallas as pl
from jax.experimental.pallas import tpu as pltpu
from jax.experimental.pallas import tpu_sc as plsc

B, T, C_IN, D_MODEL = 16, 2048, 16, 128
HALF = T // 2
MBINS = HALF // 2
CHUNK = 128
NCHUNK = MBINS // CHUNK
GB = 4
HI = jax.lax.Precision.HIGHEST
MED = jax.lax.Precision.DEFAULT


def _sinusoid_table(rows, d_model):
    pos = np.arange(rows, dtype=np.float32)[:, None]
    div = np.exp(np.arange(0, d_model, 2, dtype=np.float32)
                 * -(math.log(10000.0) / d_model))
    w = np.zeros((rows, d_model), dtype=np.float32)
    w[:, 0::2] = np.sin(pos * div)
    w[:, 1::2] = np.cos(pos * div)
    return w


_PE = _sinusoid_table(T, D_MODEL)
_TAB8 = _sinusoid_table(8, D_MODEL)
_tt = np.arange(HALF, dtype=np.float64)[:, None]
_mm = np.arange(MBINS, dtype=np.float64)[None, :]
_CO = np.cos(2.0 * np.pi * _tt * (2.0 * _mm + 1.0) / T).astype(np.float32)
_SO = np.sin(2.0 * np.pi * _tt * (2.0 * _mm + 1.0) / T).astype(np.float32)
_t2 = np.arange(HALF // 2, dtype=np.float64)[:, None]
_kk = np.arange(MBINS // 2, dtype=np.float64)[None, :]
_CEE = np.cos(2.0 * np.pi * _t2 * _kk / (HALF // 2)).astype(np.float32)
_SEE = np.sin(2.0 * np.pi * _t2 * _kk / (HALF // 2)).astype(np.float32)
_CEO = np.cos(2.0 * np.pi * _t2 * (2.0 * _kk + 1.0) / HALF).astype(np.float32)
_SEO = np.sin(2.0 * np.pi * _t2 * (2.0 * _kk + 1.0) / HALF).astype(np.float32)
_TABS = np.stack([
    np.concatenate([t[:, c * CHUNK:(c + 1) * CHUNK] for t in (_CO, _SO)],
                   axis=1)
    for c in range(NCHUNK)])
_EPACK = np.concatenate([_CEE, _SEE, _CEO, _SEO], axis=1)
_SEL = (np.arange(B)[:, None] ==
        (np.arange(B * C_IN)[None, :] // C_IN)).astype(np.float32)


_NWORKERS = 32
_TOK_PER_W = B * T // _NWORKERS
_ITERS = _TOK_PER_W // 16


def _counts_sc_body(xm_hbm, out_hbm, xm_v, cnt_v):
    wid = lax.axis_index("s") * 2 + lax.axis_index("c")
    base = wid * _TOK_PER_W
    pltpu.sync_copy(xm_hbm.at[:, pl.ds(base, _TOK_PER_W)], xm_v)
    iota = lax.broadcasted_iota(jnp.int32, (16,), 0)
    ones = jnp.full((16,), 1.0, dtype=jnp.float32)
    zeros = jnp.zeros((16,), dtype=jnp.float32)

    def body(it, _):
        tok = iota + it * 16
        for j in range(8):
            plsc.store_scatter(cnt_v, [jnp.full((16,), j, jnp.int32), tok],
                               zeros)
        for i in range(4):
            vi = plsc.load_gather(xm_v, [jnp.full((16,), i, jnp.int32), tok])
            plsc.addupdate_scatter(cnt_v, [vi, tok], ones)
        return _

    lax.fori_loop(0, _ITERS, body, 0)
    pltpu.sync_copy(cnt_v, out_hbm.at[:, pl.ds(base, _TOK_PER_W)])


@functools.cache
def _counts_sc():
    return pl.kernel(
        _counts_sc_body,
        mesh=plsc.VectorSubcoreMesh(core_axis_name="c", subcore_axis_name="s"),
        out_type=jax.ShapeDtypeStruct((8, B * T), jnp.float32),
        scratch_types=[
            pltpu.VMEM((4, _TOK_PER_W), jnp.int32),
            pltpu.VMEM((8, _TOK_PER_W), jnp.float32),
        ],
        compiler_params=pltpu.CompilerParams(needs_layout_passes=False),
    )


def _fused_body(xt_ref, tabs_ref, ep_ref, sel_ref,
                x_ref, cnt_ref, w_ref, tab8_ref, pe_ref,
                out_ref, alpha_ref, beta_ref, runmax_ref):
    c = pl.program_id(0)
    Q = HALF // 2

    @pl.when(c < NCHUNK)
    def _():
        xt = xt_ref[...]
        xe = xt[:, :HALF] + xt[:, HALF:]
        xo = xt[:, :HALF] - xt[:, HALF:]
        tabs = tabs_ref[0]
        reo = jax.lax.dot(xo, tabs[:, :CHUNK], precision=HI)
        imo = jax.lax.dot(xo, tabs[:, CHUNK:], precision=HI)
        mag2 = reo * reo + imo * imo
        chmax = jnp.max(mag2, axis=1, keepdims=True)
        chmax = jax.lax.broadcast_in_dim(chmax, (B * C_IN, D_MODEL), (0, 1))

        @pl.when(c == 0)
        def _():
            K2 = MBINS // 2
            xee = xe[:, :Q] + xe[:, Q:]
            xeo = xe[:, :Q] - xe[:, Q:]
            ep = ep_ref[...]
            ree = jax.lax.dot(xee, ep[:, :K2], precision=HI)
            ime = jax.lax.dot(xee, ep[:, K2:2 * K2], precision=HI)
            re2 = jax.lax.dot(xeo, ep[:, 2 * K2:3 * K2], precision=HI)
            im2 = jax.lax.dot(xeo, ep[:, 3 * K2:], precision=HI)
            emag = jnp.maximum(ree * ree + ime * ime, re2 * re2 + im2 * im2)
            echmax = jnp.max(emag, axis=1, keepdims=True)
            echmax = jax.lax.broadcast_in_dim(
                echmax, (B * C_IN, D_MODEL), (0, 1))
            runmax_ref[...] = jnp.maximum(chmax, echmax)

        @pl.when(c != 0)
        def _():
            runmax_ref[...] = jnp.maximum(runmax_ref[...], chmax)

        @pl.when(c == NCHUNK - 1)
        def _():
            ti = jax.lax.broadcasted_iota(jnp.int32, (1, HALF), 1)
            alt = (1 - 2 * (ti % 2)).astype(jnp.float32)
            nyqre = jnp.sum(xe * alt, axis=1, keepdims=True)
            nyq2 = nyqre * nyqre
            is_nyq = (nyq2 > runmax_ref[:, 0:1]).astype(jnp.float32)
            count = jax.lax.dot(sel_ref[...], is_nyq, precision=HI)
            beta = count * (1.0 / C_IN)
            beta_ref[...] = jax.lax.broadcast_in_dim(beta, (B, D_MODEL), (0, 1))
            alpha_ref[...] = 1.0 - beta_ref[...]

    @pl.when(c >= NCHUNK)
    def _():
        cntall = cnt_ref[...]
        pe = pe_ref[...]
        for i in range(GB):
            xb = x_ref[i]
            xprev = jnp.concatenate([xb[-1:], xb[:-1]], axis=0)
            xnext = jnp.concatenate([xb[1:], xb[:1]], axis=0)
            x3 = jnp.concatenate([xprev, xb, xnext], axis=1)
            val = jax.lax.dot(x3, w_ref[...], precision=MED)

            temporal = jax.lax.dot_general(
                cntall[:, i * T:(i + 1) * T], tab8_ref[...],
                (((0,), (0,)), ((), ())), precision=MED)

            bi = (c - NCHUNK) * GB + i
            a = alpha_ref[pl.ds(bi, 1), :]
            b0 = beta_ref[pl.ds(bi, 1), :] * pe[0:1, :]
            out_ref[i] = val + temporal + a * pe + b0


def kernel(x, x_mark, W_conv):
    xt = jnp.transpose(x, (0, 2, 1)).reshape(B * C_IN, T)
    wt = jnp.transpose(W_conv, (2, 1, 0)).reshape(3 * C_IN, D_MODEL)

    xmT = jnp.transpose(x_mark.reshape(B * T, 4))
    counts = _counts_sc()(xmT)

    out = pl.pallas_call(
        _fused_body,
        grid=(NCHUNK + B // GB,),
        in_specs=[
            pl.BlockSpec((B * C_IN, T), lambda c: (0, 0)),
            pl.BlockSpec((1, HALF, 2 * CHUNK),
                         lambda c: (jnp.minimum(c, NCHUNK - 1), 0, 0)),
            pl.BlockSpec((HALF // 2, 2 * MBINS), lambda c: (0, 0)),
            pl.BlockSpec((B, B * C_IN), lambda c: (0, 0)),
            pl.BlockSpec((GB, T, C_IN),
                         lambda c: (jnp.maximum(c - NCHUNK, 0), 0, 0)),
            pl.BlockSpec((8, GB * T),
                         lambda c: (0, jnp.maximum(c - NCHUNK, 0))),
            pl.BlockSpec((3 * C_IN, D_MODEL), lambda c: (0, 0)),
            pl.BlockSpec((8, D_MODEL), lambda c: (0, 0)),
            pl.BlockSpec((T, D_MODEL), lambda c: (0, 0)),
        ],
        out_specs=pl.BlockSpec((GB, T, D_MODEL),
                               lambda c: (jnp.maximum(c - NCHUNK, 0), 0, 0)),
        out_shape=jax.ShapeDtypeStruct((B, T, D_MODEL), jnp.float32),
        scratch_shapes=[
            pltpu.VMEM((B, D_MODEL), jnp.float32),
            pltpu.VMEM((B, D_MODEL), jnp.float32),
            pltpu.VMEM((B * C_IN, D_MODEL), jnp.float32),
        ],
    )(xt, jnp.asarray(_TABS), jnp.asarray(_EPACK), jnp.asarray(_SEL),
      x, counts, wt, jnp.asarray(_TAB8), jnp.asarray(_PE))
    return out
